# baseline (speedup 1.0000x reference)
import jax
import jax.numpy as jnp
from jax import lax
from jax.experimental import pallas as pl
from jax.experimental.pallas import tpu as pltpu
from jax.experimental.pallas import tpu_sc as plsc

NC = 2
NS = 16
NW = NC * NS
L = 16

B, S, D = 2, 8192, 4096
NWIN = 64
ROWS_TOTAL = B * S
ROWS_W = ROWS_TOTAL // NW
OUT_W = ROWS_W * NWIN
RB = 8
NCHUNK = ROWS_W // RB


def _sc_window_select(xt, w_hbm, out, w_v, vbuf0, vbuf1, obuf, sem_a, sem_b):
    wid = lax.axis_index("s") * NC + lax.axis_index("c")
    row0 = wid * ROWS_W

    pltpu.sync_copy(w_hbm, w_v)
    wvec = [w_v[pl.ds(t * L, L)] for t in range(4)]

    def dma(k, buf, sem):
        src = xt.at[pl.ds((row0 + k * RB) * D, RB * D)]
        return pltpu.make_async_copy(src, buf, sem)

    def compact(k, buf):
        for r in range(RB):
            for t in range(4):
                vals = plsc.load_gather(buf, [wvec[t] + r * D])
                obuf[pl.ds((k * RB + r) * NWIN + t * L, L)] = vals

    dma(0, vbuf0, sem_a).start()

    def super_round(h, carry):
        k0 = 2 * h
        dma(k0 + 1, vbuf1, sem_b).start()
        dma(k0, vbuf0, sem_a).wait()
        compact(k0, vbuf0)

        @pl.when(h < NCHUNK // 2 - 1)
        def _():
            dma(k0 + 2, vbuf0, sem_a).start()

        dma(k0 + 1, vbuf1, sem_b).wait()
        compact(k0 + 1, vbuf1)
        return carry

    lax.fori_loop(0, NCHUNK // 2, super_round, 0)
    pltpu.sync_copy(obuf, out.at[wid])


@jax.jit
def kernel(x, w):
    xt = x.reshape(B * S * D)
    w32 = w.astype(jnp.int32)
    run = pl.kernel(
        _sc_window_select,
        out_type=jax.ShapeDtypeStruct((NW, OUT_W), jnp.float32),
        mesh=plsc.VectorSubcoreMesh(core_axis_name="c", subcore_axis_name="s"),
        compiler_params=pltpu.CompilerParams(needs_layout_passes=False),
        scratch_types=[
            pltpu.VMEM((NWIN,), jnp.int32),
            pltpu.VMEM((RB * D,), jnp.float32),
            pltpu.VMEM((RB * D,), jnp.float32),
            pltpu.VMEM((OUT_W,), jnp.float32),
            pltpu.SemaphoreType.DMA,
            pltpu.SemaphoreType.DMA,
        ],
    )
    out = run(xt, w32)
    return out.reshape(B, S, NWIN)
